# Initial kernel scaffold; baseline (speedup 1.0000x reference)
import jax
import jax.numpy as jnp
from jax.experimental import pallas as pl


def kernel(point_cloud, W_self1, b_self1, g_self1, be_self1, W_self2, b_self2, g_self2, be_self2, W_nb1, b_nb1, g_nb1, be_nb1, W_nb2, b_nb2, g_nb2, be_nb2, out_bias):
    raise NotImplementedError("write your pallas kernel here")


# speedup vs baseline: 3.4779x; 3.4779x over previous
import functools

import jax
import jax.numpy as jnp
from jax import lax
from jax.experimental import pallas as pl
from jax.experimental.pallas import tpu as pltpu
from jax.experimental.pallas import tpu_sc as plsc

_B, _N, _K, _FI, _FO, _H = 16, 1024, 20, 3, 16, 4
_HF = _H * _FO
_EPS = 1e-3
_BLK = 128
_NBLK = (_B * _N) // _BLK
_NW = 32
_ROWS_PER_W = (_B * _N * _K) // _NW
_CH = 128
_NCH = _ROWS_PER_W // _CH


def _knn_body(pc_ref, w1_ref, idx_ref, x_ref, adj_ref):
    b = pl.program_id(0)
    pc = pc_ref[0]
    sq = jnp.sum(pc * pc, axis=1, keepdims=True)
    pcb = pc.astype(jnp.bfloat16)
    ip = lax.dot_general(pcb, pcb, (((1,), (1,)), ((), ())),
                         preferred_element_type=jnp.float32)
    inner = -2.0 * ip
    adj_ref[...] = sq + inner + sq.reshape(1, _N)

    ci = lax.broadcasted_iota(jnp.int32, (_N, _N), 1)
    kcol = lax.broadcasted_iota(jnp.int32, (_N, _K), 1)

    def body(k, out_idx):
        adj = adj_ref[...]
        m = jnp.min(adj, axis=1, keepdims=True)
        cand = jnp.where(adj == m, ci, jnp.int32(2**30))
        idxk = jnp.min(cand, axis=1)
        adj_ref[...] = jnp.where(ci == idxk[:, None], jnp.float32(jnp.inf), adj)
        return jnp.where(kcol == k, idxk[:, None], out_idx)

    out_idx = lax.fori_loop(0, _K, body, jnp.zeros((_N, _K), jnp.int32))
    idx_ref[0] = out_idx + b * _N
    x_ref[0] = jnp.dot(pc, w1_ref[...], precision=lax.Precision.HIGHEST,
                       preferred_element_type=jnp.float32)


def _run_knn(pc, w1):
    return pl.pallas_call(
        _knn_body,
        grid=(_B,),
        in_specs=[
            pl.BlockSpec((1, _N, _FI), lambda b: (b, 0, 0)),
            pl.BlockSpec((_FI, _HF), lambda b: (0, 0)),
        ],
        out_specs=[
            pl.BlockSpec((1, _N, _K), lambda b: (b, 0, 0)),
            pl.BlockSpec((1, _N, _HF), lambda b: (b, 0, 0)),
        ],
        out_shape=[
            jax.ShapeDtypeStruct((_B, _N, _K), jnp.int32),
            jax.ShapeDtypeStruct((_B, _N, _HF), jnp.float32),
        ],
        scratch_shapes=[pltpu.VMEM((_N, _N), jnp.float32)],
    )(pc, w1)


def _sc_gather(x_flat, idx):
    idx3 = idx.reshape(_NW, _NCH, _CH)
    mesh = plsc.VectorSubcoreMesh(core_axis_name="c", subcore_axis_name="s")

    @functools.partial(
        pl.kernel,
        mesh=mesh,
        compiler_params=pltpu.CompilerParams(use_tc_tiling_on_sc=False),
        out_type=jax.ShapeDtypeStruct((_B * _N * _K, _HF), jnp.float32),
        scratch_types=[
            pltpu.VMEM((_NCH, _CH), jnp.int32),
            pltpu.VMEM((_CH, _HF), jnp.float32),
            pltpu.VMEM((_CH, _HF), jnp.float32),
            pltpu.SemaphoreType.DMA,
            pltpu.SemaphoreType.DMA,
        ],
    )
    def k(idx_hbm, x_hbm, out_hbm, idx_v, buf0, buf1, sem0, sem1):
        wid = lax.axis_index("s") * 2 + lax.axis_index("c")
        base = wid * _ROWS_PER_W
        pltpu.sync_copy(idx_hbm.at[wid], idx_v)

        def body(c, _):
            c0 = 2 * c
            c1 = 2 * c + 1
            cp0 = pltpu.async_copy(x_hbm.at[idx_v.at[c0]], buf0, sem0)
            cp1 = pltpu.async_copy(x_hbm.at[idx_v.at[c1]], buf1, sem1)
            cp0.wait()
            pltpu.sync_copy(buf0, out_hbm.at[pl.ds(base + c0 * _CH, _CH)])
            cp1.wait()
            pltpu.sync_copy(buf1, out_hbm.at[pl.ds(base + c1 * _CH, _CH)])
            return 0

        lax.fori_loop(0, _NCH // 2, body, 0)

    return k(idx3, x_flat)


def _leaky(x):
    return jnp.where(x >= 0.0, x, 0.2 * x)


def _epilogue_body(pc_ref, ws1_ref, x_ref, knn_ref, ob_ref, p64_ref, p4_ref,
                   w2_ref, att_ref, mgf_ref):
    p64 = p64_ref[...]
    p4 = p4_ref[...]
    w2 = w2_ref[...]
    pc = pc_ref[...]
    xs = x_ref[...]
    knn = knn_ref[...]

    b1s, gs1, bes1 = p64[0], p64[1], p64[2]
    bnb1, gnb1, benb1 = p64[3], p64[4], p64[5]
    b2s, gs2, bes2 = p4[0], p4[1], p4[2]
    bnb2, gnb2, benb2 = p4[3], p4[4], p4[5]

    s1 = jnp.dot(pc, ws1_ref[...], precision=lax.Precision.HIGHEST,
                 preferred_element_type=jnp.float32)
    s1 = jax.nn.relu(s1 + b1s[None, :]) * gs1[None, :] + bes1[None, :]
    s1r = s1.reshape(_BLK, _H, _FO)
    s2 = jnp.sum(s1r * w2[0][None, :, :], axis=2)
    sa = jax.nn.relu(s2 + b2s[None, :]) * gs2[None, :] + bes2[None, :]

    na = jax.nn.relu(xs[:, None, :] - knn + bnb1[None, None, :])
    na = na * gnb1[None, None, :] + benb1[None, None, :]
    na4 = na.reshape(_BLK, _K, _H, _FO)
    n2 = jnp.sum(na4 * w2[1][None, None, :, :], axis=3)
    na2 = jax.nn.relu(n2 + bnb2[None, None, :]) * gnb2[None, None, :] \
        + benb2[None, None, :]

    logits = _leaky(sa[:, None, :] + na2)
    m = jnp.max(logits, axis=1, keepdims=True)
    e = jnp.exp(logits - m)
    coef = e / jnp.sum(e, axis=1, keepdims=True)

    att = jnp.sum(coef[:, :, :, None] * na4, axis=1)
    att_ref[...] = jax.nn.relu(att.reshape(_BLK, _HF) + ob_ref[...])
    mgf_ref[...] = na


def _run_epilogue(pc2, xflat, knn3, ob, ws1, p64, p4, w2):
    return pl.pallas_call(
        _epilogue_body,
        grid=(_NBLK,),
        in_specs=[
            pl.BlockSpec((_BLK, _FI), lambda i: (i, 0)),
            pl.BlockSpec((_FI, _HF), lambda i: (0, 0)),
            pl.BlockSpec((_BLK, _HF), lambda i: (i, 0)),
            pl.BlockSpec((_BLK, _K, _HF), lambda i: (i, 0, 0)),
            pl.BlockSpec((_BLK, _HF), lambda i: (i % (_N // _BLK), 0)),
            pl.BlockSpec((6, _HF), lambda i: (0, 0)),
            pl.BlockSpec((6, _H), lambda i: (0, 0)),
            pl.BlockSpec((2, _H, _FO), lambda i: (0, 0, 0)),
        ],
        out_specs=[
            pl.BlockSpec((_BLK, _HF), lambda i: (i, 0)),
            pl.BlockSpec((_BLK, _K, _HF), lambda i: (i, 0, 0)),
        ],
        out_shape=[
            jax.ShapeDtypeStruct((_B * _N, _HF), jnp.float32),
            jax.ShapeDtypeStruct((_B * _N, _K, _HF), jnp.float32),
        ],
    )(pc2, ws1, xflat, knn3, ob, p64, p4, w2)


def kernel(point_cloud, W_self1, b_self1, g_self1, be_self1, W_self2, b_self2,
           g_self2, be_self2, W_nb1, b_nb1, g_nb1, be_nb1, W_nb2, b_nb2,
           g_nb2, be_nb2, out_bias):
    f32 = jnp.float32
    inv = 1.0 / jnp.sqrt(jnp.float32(1.0 + _EPS))

    ws1 = jnp.transpose(W_self1, (1, 0, 2)).reshape(_FI, _HF).astype(f32)
    wn1 = jnp.transpose(W_nb1, (1, 0, 2)).reshape(_FI, _HF).astype(f32)
    p64 = jnp.stack([
        b_self1.reshape(_HF), (g_self1 * inv).reshape(_HF), be_self1.reshape(_HF),
        b_nb1.reshape(_HF), (g_nb1 * inv).reshape(_HF), be_nb1.reshape(_HF),
    ]).astype(f32)
    p4 = jnp.stack([
        b_self2.reshape(_H), (g_self2 * inv).reshape(_H), be_self2.reshape(_H),
        b_nb2.reshape(_H), (g_nb2 * inv).reshape(_H), be_nb2.reshape(_H),
    ]).astype(f32)
    w2 = jnp.stack([W_self2.reshape(_H, _FO), W_nb2.reshape(_H, _FO)]).astype(f32)
    ob = jnp.transpose(out_bias, (1, 2, 0, 3)).reshape(_N, _HF).astype(f32)

    idx, x = _run_knn(point_cloud, wn1)
    xflat = x.reshape(_B * _N, _HF)
    knn = _sc_gather(xflat, idx.reshape(-1))
    knn3 = knn.reshape(_B * _N, _K, _HF)
    pc2 = point_cloud.reshape(_B * _N, _FI)
    att, mgf = _run_epilogue(pc2, xflat, knn3, ob, ws1, p64, p4, w2)
    return (att.reshape(_B, _N, 1, _HF), mgf.reshape(_B, _N, _K, _HF))


# speedup vs baseline: 5.8297x; 1.6762x over previous
import functools

import jax
import jax.numpy as jnp
from jax import lax
from jax.experimental import pallas as pl
from jax.experimental.pallas import tpu as pltpu
from jax.experimental.pallas import tpu_sc as plsc

_B, _N, _K, _FI, _FO, _H = 16, 1024, 20, 3, 16, 4
_HF = _H * _FO
_EPS = 1e-3
_BLK = 128
_NBLK = (_B * _N) // _BLK
_NW = 32
_ROWS_PER_W = (_B * _N * _K) // _NW
_CH = 128
_NCH = _ROWS_PER_W // _CH


_RB = 256
_NRB = _N // _RB


def _knn_body(pcr_ref, pc_ref, w1_ref, idx_ref, x_ref):
    b = pl.program_id(0)
    pc = pc_ref[0]
    pcr = pcr_ref[0]
    sqr = jnp.sum(pcr * pcr, axis=1, keepdims=True)
    sqf = jnp.sum(pc * pc, axis=1).reshape(1, _N)
    ip = lax.dot_general(pcr.astype(jnp.bfloat16), pc.astype(jnp.bfloat16),
                         (((1,), (1,)), ((), ())),
                         preferred_element_type=jnp.float32)
    adj = sqr + (-2.0 * ip) + sqf

    ci = lax.broadcasted_iota(jnp.int32, (_RB, _N), 1)
    kcol = lax.broadcasted_iota(jnp.int32, (_RB, _K), 1)

    def body(k, carry):
        v, i, out_idx = carry
        elig = (adj > v) | ((adj == v) & (ci > i))
        work = jnp.where(elig, adj, jnp.float32(jnp.inf))
        m = jnp.min(work, axis=1, keepdims=True)
        cand = jnp.where(work == m, ci, jnp.int32(2**30))
        ii = jnp.min(cand, axis=1, keepdims=True)
        out_idx = jnp.where(kcol == k, ii, out_idx)
        return m, ii, out_idx

    _, _, out_idx = lax.fori_loop(
        0, _K, body,
        (jnp.full((_RB, 1), -jnp.inf, jnp.float32),
         jnp.full((_RB, 1), -1, jnp.int32),
         jnp.zeros((_RB, _K), jnp.int32)))
    idx_ref[0] = out_idx + b * _N
    x_ref[0] = jnp.dot(pcr, w1_ref[...], precision=lax.Precision.HIGHEST,
                       preferred_element_type=jnp.float32)


def _run_knn(pc, w1):
    return pl.pallas_call(
        _knn_body,
        grid=(_B, _NRB),
        in_specs=[
            pl.BlockSpec((1, _RB, _FI), lambda b, r: (b, r, 0)),
            pl.BlockSpec((1, _N, _FI), lambda b, r: (b, 0, 0)),
            pl.BlockSpec((_FI, _HF), lambda b, r: (0, 0)),
        ],
        out_specs=[
            pl.BlockSpec((1, _RB, _K), lambda b, r: (b * _NRB + r, 0, 0)),
            pl.BlockSpec((1, _RB, _HF), lambda b, r: (b * _NRB + r, 0, 0)),
        ],
        out_shape=[
            jax.ShapeDtypeStruct((_B * _NRB, _RB, _K), jnp.int32),
            jax.ShapeDtypeStruct((_B * _NRB, _RB, _HF), jnp.float32),
        ],
    )(pc, pc, w1)


def _sc_gather(x_flat, idx):
    idx3 = idx.reshape(_NW, _NCH, _CH)
    mesh = plsc.VectorSubcoreMesh(core_axis_name="c", subcore_axis_name="s")

    @functools.partial(
        pl.kernel,
        mesh=mesh,
        compiler_params=pltpu.CompilerParams(use_tc_tiling_on_sc=False),
        out_type=jax.ShapeDtypeStruct((_B * _N * _K, _HF), jnp.float32),
        scratch_types=[
            pltpu.VMEM((_NCH, _CH), jnp.int32),
            pltpu.VMEM((_CH, _HF), jnp.float32),
            pltpu.VMEM((_CH, _HF), jnp.float32),
            pltpu.SemaphoreType.DMA,
            pltpu.SemaphoreType.DMA,
        ],
    )
    def k(idx_hbm, x_hbm, out_hbm, idx_v, buf0, buf1, sem0, sem1):
        wid = lax.axis_index("s") * 2 + lax.axis_index("c")
        base = wid * _ROWS_PER_W
        pltpu.sync_copy(idx_hbm.at[wid], idx_v)

        def body(c, _):
            c0 = 2 * c
            c1 = 2 * c + 1
            cp0 = pltpu.async_copy(x_hbm.at[idx_v.at[c0]], buf0, sem0)
            cp1 = pltpu.async_copy(x_hbm.at[idx_v.at[c1]], buf1, sem1)
            cp0.wait()
            pltpu.sync_copy(buf0, out_hbm.at[pl.ds(base + c0 * _CH, _CH)])
            cp1.wait()
            pltpu.sync_copy(buf1, out_hbm.at[pl.ds(base + c1 * _CH, _CH)])
            return 0

        lax.fori_loop(0, _NCH // 2, body, 0)

    return k(idx3, x_flat)


def _leaky(x):
    return jnp.where(x >= 0.0, x, 0.2 * x)


def _epilogue_body(pc_ref, ws1_ref, x_ref, knn_ref, ob_ref, p64_ref, pe_ref,
                   wrow_ref, s64_ref, att_ref, mgf_ref):
    p64 = p64_ref[...]
    pe = pe_ref[...]
    wrow = wrow_ref[...]
    sseg = s64_ref[...]
    pc = pc_ref[...]
    xs = x_ref[...]
    knn = knn_ref[...]

    b1s, gs1, bes1 = p64[0], p64[1], p64[2]
    bnb1, gnb1, benb1 = p64[3], p64[4], p64[5]
    b2e, g2e, be2e = pe[0], pe[1], pe[2]
    bnb2e, gnb2e, benb2e = pe[3], pe[4], pe[5]

    s1 = jnp.dot(pc, ws1_ref[...], precision=lax.Precision.HIGHEST,
                 preferred_element_type=jnp.float32)
    s1 = jax.nn.relu(s1 + b1s[None, :]) * gs1[None, :] + bes1[None, :]
    s2e = jnp.dot((s1 * wrow[0][None, :]).astype(jnp.bfloat16), sseg,
                  preferred_element_type=jnp.float32)
    sa_e = jax.nn.relu(s2e + b2e[None, :]) * g2e[None, :] + be2e[None, :]

    na = jax.nn.relu(xs[:, None, :] - knn + bnb1[None, None, :])
    na = na * gnb1[None, None, :] + benb1[None, None, :]
    pn = (na * wrow[1][None, None, :]).reshape(_BLK * _K, _HF)
    na2e = jnp.dot(pn.astype(jnp.bfloat16), sseg,
                   preferred_element_type=jnp.float32).reshape(_BLK, _K, _HF)
    na2b = jax.nn.relu(na2e + bnb2e[None, None, :]) * gnb2e[None, None, :] \
        + benb2e[None, None, :]

    lg = _leaky(sa_e[:, None, :] + na2b)
    m = jnp.max(lg, axis=1, keepdims=True)
    e = jnp.exp(lg - m)
    coef = e / jnp.sum(e, axis=1, keepdims=True)

    att = jnp.sum(coef * na, axis=1)
    att_ref[...] = jax.nn.relu(att + ob_ref[...])
    mgf_ref[...] = na


def _run_epilogue(pc2, xflat, knn3, ob, ws1, p64, pe, wrow, s64):
    return pl.pallas_call(
        _epilogue_body,
        grid=(_NBLK,),
        in_specs=[
            pl.BlockSpec((_BLK, _FI), lambda i: (i, 0)),
            pl.BlockSpec((_FI, _HF), lambda i: (0, 0)),
            pl.BlockSpec((_BLK, _HF), lambda i: (i, 0)),
            pl.BlockSpec((_BLK, _K, _HF), lambda i: (i, 0, 0)),
            pl.BlockSpec((_BLK, _HF), lambda i: (i % (_N // _BLK), 0)),
            pl.BlockSpec((6, _HF), lambda i: (0, 0)),
            pl.BlockSpec((6, _HF), lambda i: (0, 0)),
            pl.BlockSpec((2, _HF), lambda i: (0, 0)),
            pl.BlockSpec((_HF, _HF), lambda i: (0, 0)),
        ],
        out_specs=[
            pl.BlockSpec((_BLK, _HF), lambda i: (i, 0)),
            pl.BlockSpec((_BLK, _K, _HF), lambda i: (i, 0, 0)),
        ],
        out_shape=[
            jax.ShapeDtypeStruct((_B * _N, _HF), jnp.float32),
            jax.ShapeDtypeStruct((_B * _N, _K, _HF), jnp.float32),
        ],
    )(pc2, ws1, xflat, knn3, ob, p64, pe, wrow, s64)


def kernel(point_cloud, W_self1, b_self1, g_self1, be_self1, W_self2, b_self2,
           g_self2, be_self2, W_nb1, b_nb1, g_nb1, be_nb1, W_nb2, b_nb2,
           g_nb2, be_nb2, out_bias):
    f32 = jnp.float32
    inv = 1.0 / jnp.sqrt(jnp.float32(1.0 + _EPS))

    ws1 = jnp.transpose(W_self1, (1, 0, 2)).reshape(_FI, _HF).astype(f32)
    wn1 = jnp.transpose(W_nb1, (1, 0, 2)).reshape(_FI, _HF).astype(f32)
    p64 = jnp.stack([
        b_self1.reshape(_HF), (g_self1 * inv).reshape(_HF), be_self1.reshape(_HF),
        b_nb1.reshape(_HF), (g_nb1 * inv).reshape(_HF), be_nb1.reshape(_HF),
    ]).astype(f32)
    def expand(v):
        return jnp.broadcast_to(v.reshape(_H, 1), (_H, _FO)).reshape(_HF)

    pe = jnp.stack([
        expand(b_self2), expand(g_self2 * inv), expand(be_self2),
        expand(b_nb2), expand(g_nb2 * inv), expand(be_nb2),
    ]).astype(f32)
    wrow = jnp.stack([W_self2.reshape(_HF), W_nb2.reshape(_HF)]).astype(f32)
    seg = jnp.arange(_HF, dtype=jnp.int32) // _FO
    s64 = (seg[:, None] == seg[None, :]).astype(jnp.bfloat16)
    ob = jnp.transpose(out_bias, (1, 2, 0, 3)).reshape(_N, _HF).astype(f32)

    idx, x = _run_knn(point_cloud, wn1)
    xflat = x.reshape(_B * _N, _HF)
    knn = _sc_gather(xflat, idx.reshape(-1))
    knn3 = knn.reshape(_B * _N, _K, _HF)
    pc2 = point_cloud.reshape(_B * _N, _FI)
    att, mgf = _run_epilogue(pc2, xflat, knn3, ob, ws1, p64, pe, wrow, s64)
    return (att.reshape(_B, _N, 1, _HF), mgf.reshape(_B, _N, _K, _HF))


# speedup vs baseline: 7.8178x; 1.3410x over previous
import functools

import jax
import jax.numpy as jnp
from jax import lax
from jax.experimental import pallas as pl
from jax.experimental.pallas import tpu as pltpu
from jax.experimental.pallas import tpu_sc as plsc

_B, _N, _K, _FI, _FO, _H = 16, 1024, 20, 3, 16, 4
_HF = _H * _FO
_EPS = 1e-3
_BLK = 128
_NBLK = (_B * _N) // _BLK
_NW = 32
_ROWS_PER_W = (_B * _N * _K) // _NW
_CH = 128
_NCH = _ROWS_PER_W // _CH


_RB = 256
_NRB = _N // _RB


def _knn_body(pc_ref, w1_ref, idx_ref, x_ref, adj_ref):
    b = pl.program_id(0)
    pc = pc_ref[0]
    sq = jnp.sum(pc * pc, axis=1, keepdims=True)
    pcb = pc.astype(jnp.bfloat16)
    ip = lax.dot_general(pcb, pcb, (((1,), (1,)), ((), ())),
                         preferred_element_type=jnp.float32)
    adj_ref[...] = sq + (-2.0 * ip) + sq.reshape(1, _N)

    ci = lax.broadcasted_iota(jnp.int32, (_N, _N), 1)
    kcol = lax.broadcasted_iota(jnp.int32, (_N, _K), 1)

    def body(k, out_idx):
        adj = adj_ref[...]
        m = jnp.min(adj, axis=1, keepdims=True)
        cand = jnp.where(adj == m, ci, jnp.int32(2**30))
        idxk = jnp.min(cand, axis=1)
        adj_ref[...] = jnp.where(ci == idxk[:, None], jnp.float32(jnp.inf), adj)
        return jnp.where(kcol == k, idxk[:, None], out_idx)

    out_idx = lax.fori_loop(0, _K, body, jnp.zeros((_N, _K), jnp.int32))
    idx_ref[0] = out_idx + b * _N
    x_ref[0] = jnp.dot(pc, w1_ref[...], precision=lax.Precision.HIGHEST,
                       preferred_element_type=jnp.float32)


def _run_knn(pc, w1):
    return pl.pallas_call(
        _knn_body,
        grid=(_B,),
        in_specs=[
            pl.BlockSpec((1, _N, _FI), lambda b: (b, 0, 0)),
            pl.BlockSpec((_FI, _HF), lambda b: (0, 0)),
        ],
        out_specs=[
            pl.BlockSpec((1, _N, _K), lambda b: (b, 0, 0)),
            pl.BlockSpec((1, _N, _HF), lambda b: (b, 0, 0)),
        ],
        out_shape=[
            jax.ShapeDtypeStruct((_B, _N, _K), jnp.int32),
            jax.ShapeDtypeStruct((_B, _N, _HF), jnp.float32),
        ],
        scratch_shapes=[pltpu.VMEM((_N, _N), jnp.float32)],
    )(pc, w1)


def _sc_gather(x_flat, idx):
    idx3 = idx.reshape(_NW, _NCH, _CH)
    mesh = plsc.VectorSubcoreMesh(core_axis_name="c", subcore_axis_name="s")

    @functools.partial(
        pl.kernel,
        mesh=mesh,
        compiler_params=pltpu.CompilerParams(use_tc_tiling_on_sc=False),
        out_type=jax.ShapeDtypeStruct((_B * _N * _K, _HF), jnp.float32),
        scratch_types=[
            pltpu.VMEM((_NCH, _CH), jnp.int32),
            pltpu.VMEM((_CH, _HF), jnp.float32),
            pltpu.VMEM((_CH, _HF), jnp.float32),
            pltpu.SemaphoreType.DMA,
            pltpu.SemaphoreType.DMA,
        ],
    )
    def k(idx_hbm, x_hbm, out_hbm, idx_v, buf0, buf1, sem0, sem1):
        wid = lax.axis_index("s") * 2 + lax.axis_index("c")
        base = wid * _ROWS_PER_W
        pltpu.sync_copy(idx_hbm.at[wid], idx_v)

        def body(c, _):
            c0 = 2 * c
            c1 = 2 * c + 1
            cp0 = pltpu.async_copy(x_hbm.at[idx_v.at[c0]], buf0, sem0)
            cp1 = pltpu.async_copy(x_hbm.at[idx_v.at[c1]], buf1, sem1)
            cp0.wait()
            pltpu.sync_copy(buf0, out_hbm.at[pl.ds(base + c0 * _CH, _CH)])
            cp1.wait()
            pltpu.sync_copy(buf1, out_hbm.at[pl.ds(base + c1 * _CH, _CH)])
            return 0

        lax.fori_loop(0, _NCH // 2, body, 0)

    return k(idx3, x_flat)


def _leaky(x):
    return jnp.where(x >= 0.0, x, 0.2 * x)


def _epilogue_body(pc_ref, ws1_ref, x_ref, knn_ref, ob_ref, p64_ref, pe_ref,
                   wrow_ref, s64_ref, att_ref, mgf_ref):
    p64 = p64_ref[...]
    pe = pe_ref[...]
    wrow = wrow_ref[...]
    sseg = s64_ref[...]
    pc = pc_ref[...]
    xs = x_ref[...]
    knn = knn_ref[...]

    b1s, gs1, bes1 = p64[0], p64[1], p64[2]
    bnb1, gnb1, benb1 = p64[3], p64[4], p64[5]
    b2e, g2e, be2e = pe[0], pe[1], pe[2]
    bnb2e, gnb2e, benb2e = pe[3], pe[4], pe[5]

    s1 = jnp.dot(pc, ws1_ref[...], precision=lax.Precision.HIGHEST,
                 preferred_element_type=jnp.float32)
    s1 = jax.nn.relu(s1 + b1s[None, :]) * gs1[None, :] + bes1[None, :]
    s2e = jnp.dot((s1 * wrow[0][None, :]).astype(jnp.bfloat16), sseg,
                  preferred_element_type=jnp.float32)
    sa_e = jax.nn.relu(s2e + b2e[None, :]) * g2e[None, :] + be2e[None, :]

    na = jax.nn.relu(xs[:, None, :] - knn + bnb1[None, None, :])
    na = na * gnb1[None, None, :] + benb1[None, None, :]
    pn = (na * wrow[1][None, None, :]).reshape(_BLK * _K, _HF)
    na2e = jnp.dot(pn.astype(jnp.bfloat16), sseg,
                   preferred_element_type=jnp.float32).reshape(_BLK, _K, _HF)
    na2b = jax.nn.relu(na2e + bnb2e[None, None, :]) * gnb2e[None, None, :] \
        + benb2e[None, None, :]

    lg = _leaky(sa_e[:, None, :] + na2b)
    m = jnp.max(lg, axis=1, keepdims=True)
    e = jnp.exp(lg - m)
    coef = e / jnp.sum(e, axis=1, keepdims=True)

    att = jnp.sum(coef * na, axis=1)
    att_ref[...] = jax.nn.relu(att + ob_ref[...])
    mgf_ref[...] = na


def _run_epilogue(pc2, xflat, knn3, ob, ws1, p64, pe, wrow, s64):
    return pl.pallas_call(
        _epilogue_body,
        grid=(_NBLK,),
        in_specs=[
            pl.BlockSpec((_BLK, _FI), lambda i: (i, 0)),
            pl.BlockSpec((_FI, _HF), lambda i: (0, 0)),
            pl.BlockSpec((_BLK, _HF), lambda i: (i, 0)),
            pl.BlockSpec((_BLK, _K, _HF), lambda i: (i, 0, 0)),
            pl.BlockSpec((_BLK, _HF), lambda i: (i % (_N // _BLK), 0)),
            pl.BlockSpec((6, _HF), lambda i: (0, 0)),
            pl.BlockSpec((6, _HF), lambda i: (0, 0)),
            pl.BlockSpec((2, _HF), lambda i: (0, 0)),
            pl.BlockSpec((_HF, _HF), lambda i: (0, 0)),
        ],
        out_specs=[
            pl.BlockSpec((_BLK, _HF), lambda i: (i, 0)),
            pl.BlockSpec((_BLK, _K, _HF), lambda i: (i, 0, 0)),
        ],
        out_shape=[
            jax.ShapeDtypeStruct((_B * _N, _HF), jnp.float32),
            jax.ShapeDtypeStruct((_B * _N, _K, _HF), jnp.float32),
        ],
    )(pc2, ws1, xflat, knn3, ob, p64, pe, wrow, s64)


def kernel(point_cloud, W_self1, b_self1, g_self1, be_self1, W_self2, b_self2,
           g_self2, be_self2, W_nb1, b_nb1, g_nb1, be_nb1, W_nb2, b_nb2,
           g_nb2, be_nb2, out_bias):
    f32 = jnp.float32
    inv = 1.0 / jnp.sqrt(jnp.float32(1.0 + _EPS))

    ws1 = jnp.transpose(W_self1, (1, 0, 2)).reshape(_FI, _HF).astype(f32)
    wn1 = jnp.transpose(W_nb1, (1, 0, 2)).reshape(_FI, _HF).astype(f32)
    p64 = jnp.stack([
        b_self1.reshape(_HF), (g_self1 * inv).reshape(_HF), be_self1.reshape(_HF),
        b_nb1.reshape(_HF), (g_nb1 * inv).reshape(_HF), be_nb1.reshape(_HF),
    ]).astype(f32)
    def expand(v):
        return jnp.broadcast_to(v.reshape(_H, 1), (_H, _FO)).reshape(_HF)

    pe = jnp.stack([
        expand(b_self2), expand(g_self2 * inv), expand(be_self2),
        expand(b_nb2), expand(g_nb2 * inv), expand(be_nb2),
    ]).astype(f32)
    wrow = jnp.stack([W_self2.reshape(_HF), W_nb2.reshape(_HF)]).astype(f32)
    seg = jnp.arange(_HF, dtype=jnp.int32) // _FO
    s64 = (seg[:, None] == seg[None, :]).astype(jnp.bfloat16)
    ob = jnp.transpose(out_bias, (1, 2, 0, 3)).reshape(_N, _HF).astype(f32)

    idx, x = _run_knn(point_cloud, wn1)
    xflat = x.reshape(_B * _N, _HF)
    knn = _sc_gather(xflat, idx.reshape(-1))
    knn3 = knn.reshape(_B * _N, _K, _HF)
    pc2 = point_cloud.reshape(_B * _N, _FI)
    att, mgf = _run_epilogue(pc2, xflat, knn3, ob, ws1, p64, pe, wrow, s64)
    return (att.reshape(_B, _N, 1, _HF), mgf.reshape(_B, _N, _K, _HF))


# speedup vs baseline: 7.8292x; 1.0015x over previous
import functools

import jax
import jax.numpy as jnp
from jax import lax
from jax.experimental import pallas as pl
from jax.experimental.pallas import tpu as pltpu
from jax.experimental.pallas import tpu_sc as plsc

_B, _N, _K, _FI, _FO, _H = 16, 1024, 20, 3, 16, 4
_HF = _H * _FO
_EPS = 1e-3
_BLK = 128
_NBLK = (_B * _N) // _BLK
_NW = 32
_ROWS_PER_W = (_B * _N * _K) // _NW
_CH = 128
_NCH = _ROWS_PER_W // _CH


def _knn_body(pc_ref, w1_ref, idx_ref, x_ref, adj_ref):
    b = pl.program_id(0)
    pc = pc_ref[0]
    sq = jnp.sum(pc * pc, axis=1, keepdims=True)
    pcb = pc.astype(jnp.bfloat16)
    ip = lax.dot_general(pcb, pcb, (((1,), (1,)), ((), ())),
                         preferred_element_type=jnp.float32)
    adj_ref[...] = sq + (-2.0 * ip) + sq.reshape(1, _N)

    ci = lax.broadcasted_iota(jnp.int32, (_N, _N), 1)
    kcol = lax.broadcasted_iota(jnp.int32, (_N, _K), 1)

    def body(k, out_idx):
        adj = adj_ref[...]
        m = jnp.min(adj, axis=1, keepdims=True)
        cand = jnp.where(adj == m, ci, jnp.int32(2**30))
        idxk = jnp.min(cand, axis=1)
        adj_ref[...] = jnp.where(ci == idxk[:, None], jnp.float32(jnp.inf), adj)
        return jnp.where(kcol == k, idxk[:, None], out_idx)

    out_idx = lax.fori_loop(0, _K, body, jnp.zeros((_N, _K), jnp.int32))
    idx_ref[0] = out_idx + b * _N
    x_ref[0] = jnp.dot(pc, w1_ref[...], precision=lax.Precision.HIGHEST,
                       preferred_element_type=jnp.float32)


def _run_knn(pc, w1):
    return pl.pallas_call(
        _knn_body,
        grid=(_B,),
        in_specs=[
            pl.BlockSpec((1, _N, _FI), lambda b: (b, 0, 0)),
            pl.BlockSpec((_FI, _HF), lambda b: (0, 0)),
        ],
        out_specs=[
            pl.BlockSpec((1, _N, _K), lambda b: (b, 0, 0)),
            pl.BlockSpec((1, _N, _HF), lambda b: (b, 0, 0)),
        ],
        out_shape=[
            jax.ShapeDtypeStruct((_B, _N, _K), jnp.int32),
            jax.ShapeDtypeStruct((_B, _N, _HF), jnp.float32),
        ],
        scratch_shapes=[pltpu.VMEM((_N, _N), jnp.float32)],
    )(pc, w1)


def _sc_gather(x_flat, idx):
    idx3 = idx.reshape(_NW, _NCH, _CH)
    mesh = plsc.VectorSubcoreMesh(core_axis_name="c", subcore_axis_name="s")

    @functools.partial(
        pl.kernel,
        mesh=mesh,
        compiler_params=pltpu.CompilerParams(use_tc_tiling_on_sc=False),
        out_type=jax.ShapeDtypeStruct((_B * _N * _K, _HF), jnp.float32),
        scratch_types=[
            pltpu.VMEM((_NCH, _CH), jnp.int32),
            pltpu.VMEM((_CH, _HF), jnp.float32),
            pltpu.VMEM((_CH, _HF), jnp.float32),
            pltpu.SemaphoreType.DMA,
            pltpu.SemaphoreType.DMA,
        ],
    )
    def k(idx_hbm, x_hbm, out_hbm, idx_v, buf0, buf1, sem0, sem1):
        wid = lax.axis_index("s") * 2 + lax.axis_index("c")
        base = wid * _ROWS_PER_W
        pltpu.sync_copy(idx_hbm.at[wid], idx_v)

        def body(c, _):
            c0 = 2 * c
            c1 = 2 * c + 1
            cp0 = pltpu.async_copy(x_hbm.at[idx_v.at[c0]], buf0, sem0)
            cp1 = pltpu.async_copy(x_hbm.at[idx_v.at[c1]], buf1, sem1)
            cp0.wait()
            pltpu.sync_copy(buf0, out_hbm.at[pl.ds(base + c0 * _CH, _CH)])
            cp1.wait()
            pltpu.sync_copy(buf1, out_hbm.at[pl.ds(base + c1 * _CH, _CH)])
            return 0

        lax.fori_loop(0, _NCH // 2, body, 0)

    return k(idx3, x_flat)


def _leaky(x):
    return jnp.where(x >= 0.0, x, 0.2 * x)


def _epilogue_body(pc_ref, ws1_ref, x_ref, knn_ref, ob_ref, p64_ref, pe_ref,
                   wrow_ref, s64_ref, att_ref, mgf_ref):
    p64 = p64_ref[...]
    pe = pe_ref[...]
    wrow = wrow_ref[...]
    sseg = s64_ref[...]
    pc = pc_ref[...]
    xs = x_ref[...]
    knn = knn_ref[...]

    b1s, gs1, bes1 = p64[0], p64[1], p64[2]
    bnb1, gnb1, benb1 = p64[3], p64[4], p64[5]
    b2e, g2e, be2e = pe[0], pe[1], pe[2]
    bnb2e, gnb2e, benb2e = pe[3], pe[4], pe[5]

    s1 = jnp.dot(pc, ws1_ref[...], precision=lax.Precision.HIGHEST,
                 preferred_element_type=jnp.float32)
    s1 = jax.nn.relu(s1 + b1s[None, :]) * gs1[None, :] + bes1[None, :]
    s2e = jnp.dot((s1 * wrow[0][None, :]).astype(jnp.bfloat16), sseg,
                  preferred_element_type=jnp.float32)
    sa_e = jax.nn.relu(s2e + b2e[None, :]) * g2e[None, :] + be2e[None, :]

    na = jax.nn.relu(xs[:, None, :] - knn + bnb1[None, None, :])
    na = na * gnb1[None, None, :] + benb1[None, None, :]
    pn = (na * wrow[1][None, None, :]).reshape(_BLK * _K, _HF)
    na2e = jnp.dot(pn.astype(jnp.bfloat16), sseg,
                   preferred_element_type=jnp.float32).reshape(_BLK, _K, _HF)
    na2b = jax.nn.relu(na2e + bnb2e[None, None, :]) * gnb2e[None, None, :] \
        + benb2e[None, None, :]

    lg = _leaky(sa_e[:, None, :] + na2b)
    m = jnp.max(lg, axis=1, keepdims=True)
    e = jnp.exp(lg - m)
    coef = e / jnp.sum(e, axis=1, keepdims=True)

    att = jnp.sum(coef * na, axis=1)
    att_ref[...] = jax.nn.relu(att + ob_ref[...])
    mgf_ref[...] = na


def _run_epilogue(pc2, xflat, knn3, ob, ws1, p64, pe, wrow, s64):
    return pl.pallas_call(
        _epilogue_body,
        grid=(_NBLK,),
        in_specs=[
            pl.BlockSpec((_BLK, _FI), lambda i: (i, 0)),
            pl.BlockSpec((_FI, _HF), lambda i: (0, 0)),
            pl.BlockSpec((_BLK, _HF), lambda i: (i, 0)),
            pl.BlockSpec((_BLK, _K, _HF), lambda i: (i, 0, 0)),
            pl.BlockSpec((_BLK, _HF), lambda i: (i % (_N // _BLK), 0)),
            pl.BlockSpec((6, _HF), lambda i: (0, 0)),
            pl.BlockSpec((6, _HF), lambda i: (0, 0)),
            pl.BlockSpec((2, _HF), lambda i: (0, 0)),
            pl.BlockSpec((_HF, _HF), lambda i: (0, 0)),
        ],
        out_specs=[
            pl.BlockSpec((_BLK, _HF), lambda i: (i, 0)),
            pl.BlockSpec((_BLK, _K, _HF), lambda i: (i, 0, 0)),
        ],
        out_shape=[
            jax.ShapeDtypeStruct((_B * _N, _HF), jnp.float32),
            jax.ShapeDtypeStruct((_B * _N, _K, _HF), jnp.float32),
        ],
    )(pc2, ws1, xflat, knn3, ob, p64, pe, wrow, s64)


def kernel(point_cloud, W_self1, b_self1, g_self1, be_self1, W_self2, b_self2,
           g_self2, be_self2, W_nb1, b_nb1, g_nb1, be_nb1, W_nb2, b_nb2,
           g_nb2, be_nb2, out_bias):
    f32 = jnp.float32
    inv = 1.0 / jnp.sqrt(jnp.float32(1.0 + _EPS))

    ws1 = jnp.transpose(W_self1, (1, 0, 2)).reshape(_FI, _HF).astype(f32)
    wn1 = jnp.transpose(W_nb1, (1, 0, 2)).reshape(_FI, _HF).astype(f32)
    p64 = jnp.stack([
        b_self1.reshape(_HF), (g_self1 * inv).reshape(_HF), be_self1.reshape(_HF),
        b_nb1.reshape(_HF), (g_nb1 * inv).reshape(_HF), be_nb1.reshape(_HF),
    ]).astype(f32)
    def expand(v):
        return jnp.broadcast_to(v.reshape(_H, 1), (_H, _FO)).reshape(_HF)

    pe = jnp.stack([
        expand(b_self2), expand(g_self2 * inv), expand(be_self2),
        expand(b_nb2), expand(g_nb2 * inv), expand(be_nb2),
    ]).astype(f32)
    wrow = jnp.stack([W_self2.reshape(_HF), W_nb2.reshape(_HF)]).astype(f32)
    seg = jnp.arange(_HF, dtype=jnp.int32) // _FO
    s64 = (seg[:, None] == seg[None, :]).astype(jnp.bfloat16)
    ob = jnp.transpose(out_bias, (1, 2, 0, 3)).reshape(_N, _HF).astype(f32)

    idx, x = _run_knn(point_cloud, wn1)
    xflat = x.reshape(_B * _N, _HF)
    knn = _sc_gather(xflat, idx.reshape(-1))
    knn3 = knn.reshape(_B * _N, _K, _HF)
    pc2 = point_cloud.reshape(_B * _N, _FI)
    att, mgf = _run_epilogue(pc2, xflat, knn3, ob, ws1, p64, pe, wrow, s64)
    return (att.reshape(_B, _N, 1, _HF), mgf.reshape(_B, _N, _K, _HF))
